# Initial kernel scaffold; baseline (speedup 1.0000x reference)
import jax
import jax.numpy as jnp
from jax.experimental import pallas as pl


def kernel(points):
    raise NotImplementedError("write your pallas kernel here")


# speedup vs baseline: 4.5498x; 4.5498x over previous
import functools

import jax
import jax.numpy as jnp
import numpy as np
from jax.experimental import pallas as pl
from jax.experimental.pallas import tpu as pltpu, tpu_sc as plsc

GX, GY, GZ = 100, 100, 10
NCELL = GX * GY * GZ
SENT = NCELL
MAXV = 20000
MAXP = 32
NP_ = 300000
NPP = 300032
VSX = np.float32(0.01)
VSY = np.float32(0.01)
VSZ = np.float32(0.1)

TH = 100352
CR = 6272
CRQ = 1568
CH = 18752
A_SUBS = [1024] * 18 + [320]
NBLK = NPP // 1024
RTRASH = 2048
UNIQ_LEN = 22528
UPT = UNIQ_LEN // 16
TRASH = 2048
CLEN = 3 * (MAXV + TRASH)
CPT = 3760

BCH = 9376
B_SUBS = [512] * 18 + [160]
VROWS = MAXV * MAXP
VTRASH = 2048
VOXF = (VROWS + VTRASH) * 4

_MESH = dict(core_axis_name="c", subcore_axis_name="s")
_CP = pltpu.CompilerParams(needs_layout_passes=False)
_IOTA = lambda: jax.lax.iota(jnp.int32, 16)


def _cells_of(px, py, pz, gidx):
    cx = (px / VSX).astype(jnp.int32)
    cy = (py / VSY).astype(jnp.int32)
    cz = (pz / VSZ).astype(jnp.int32)
    valid = ((cx >= 0) & (cx < GX) & (cy >= 0) & (cy < GY)
             & (cz >= 0) & (cz < GZ) & (gidx < NP_)
             & (px >= 0) & (py >= 0) & (pz >= 0))
    flat = cz * (GX * GY) + cy * GX + cx
    return jnp.where(valid, flat, jnp.int32(SENT))


def _kernel_a(pts_hbm, cells_hbm, ranks_hbm, slot_hbm, uniq_hbm, npts_hbm,
              coors_hbm, voxnum_hbm, hist, cells_sh, sh_p, sem):
    cid = jax.lax.axis_index("c")
    sid = jax.lax.axis_index("s")

    @pl.when(cid == 0)
    def _body():
        t = sid
        iota = _IOTA()

        def p0(i, _):
            hist[pl.ds(i * 16, 16)] = jnp.zeros((16,), jnp.int32)
            return 0
        jax.lax.fori_loop(0, CR // 16, p0, 0)

        def p0b(ini):
            def fill(i, v):
                ini[pl.ds(i * 16, 16)] = jnp.full((16,), v, jnp.int32)
                return v
            jax.lax.fori_loop(0, UPT // 16, fill, jnp.int32(SENT))
            pltpu.sync_copy(ini.at[pl.ds(0, UPT)],
                            uniq_hbm.at[pl.ds(t * UPT, UPT)])
            jax.lax.fori_loop(0, UPT // 16, fill, jnp.int32(0))
            pltpu.sync_copy(ini.at[pl.ds(0, UPT)],
                            npts_hbm.at[pl.ds(t * UPT, UPT)])
        pl.run_scoped(p0b, pltpu.VMEM((UPT,), jnp.int32))

        def p0c(ini):
            def fill(i, _):
                ini[pl.ds(i * 16, 16)] = jnp.full((16,), -1, jnp.int32)
                return 0
            jax.lax.fori_loop(0, CPT // 16, fill, 0)
            pltpu.sync_copy(ini, coors_hbm.at[pl.ds(t * CPT, CPT)])
        pl.run_scoped(p0c, pltpu.VMEM((CPT,), jnp.int32))

        def p1(pts, cbuf):
            off = 0
            for n in A_SUBS:
                start = t * CH + off
                pltpu.sync_copy(pts_hbm.at[pl.ds(start * 4, n * 4)],
                                pts.at[pl.ds(0, n * 4)])

                def vloop(v, _):
                    rows = (v * 16 + iota) * 4
                    px = plsc.load_gather(pts, [rows])
                    py = plsc.load_gather(pts, [rows + 1])
                    pz = plsc.load_gather(pts, [rows + 2])
                    gidx = start + v * 16 + iota
                    cbuf[pl.ds(v * 16, 16)] = _cells_of(px, py, pz, gidx)
                    return 0
                jax.lax.fori_loop(0, n // 16, vloop, 0)
                pltpu.sync_copy(cbuf.at[pl.ds(0, n)],
                                cells_sh.at[pl.ds(start, n)])
                pltpu.sync_copy(cbuf.at[pl.ds(0, n)],
                                cells_hbm.at[pl.ds(start, n)])
                off += n
        pl.run_scoped(p1, pltpu.VMEM((4096,), jnp.float32),
                      pltpu.VMEM((1024,), jnp.int32))
        plsc.subcore_barrier()

        lo = t * CR

        def p2(cbuf, idx2, val2):
            for r in range(8):
                for kk in range(8):
                    dump = NPP + ((r * 128 + kk * 16 + iota + t * 131)
                                  & (RTRASH - 1))
                    idx2[r, pl.ds(kk * 16, 16)] = dump
                    val2[r, pl.ds(kk * 16, 16)] = jnp.zeros((16,), jnp.int32)

            def blk(s, _):
                base = s * 1024
                pltpu.sync_copy(cells_sh.at[pl.ds(base, 1024)], cbuf)

                def vec(v, nloc):
                    ids = cbuf[pl.ds(v * 16, 16)]
                    inr = (ids >= lo) & (ids < lo + CR)
                    rel = jnp.where(inr, ids - lo, 0)
                    old = plsc.load_gather(hist, [rel])
                    cnt, last = plsc.scan_count(ids)
                    plsc.addupdate_scatter(hist, [rel], cnt, mask=last & inr)
                    rank = old + cnt - 1
                    inr_i = inr.astype(jnp.int32)
                    kpref = plsc.cumsum(inr_i)
                    pos = nloc + kpref - inr_i
                    plsc.store_scatter(val2, [pos // 128, pos % 128], rank,
                                       mask=inr)
                    plsc.store_scatter(idx2, [pos // 128, pos % 128],
                                       base + v * 16 + iota, mask=inr)
                    return nloc + jnp.sum(inr_i, axis=0)
                nloc = jax.lax.fori_loop(0, 64, vec, jnp.int32(0))
                for r in range(8):
                    @pl.when(nloc > r * 128)
                    def _(r=r):
                        pltpu.async_copy(val2.at[r],
                                         ranks_hbm.at[idx2.at[r]], sem).wait()
                return 0
            jax.lax.fori_loop(0, NBLK, blk, 0)
        pl.run_scoped(p2, pltpu.VMEM((1024,), jnp.int32),
                      pltpu.VMEM((8, 128), jnp.int32),
                      pltpu.VMEM((8, 128), jnp.int32))

        def p3a(pbuf, small):
            def occv(v, acc):
                cellid = lo + v * 16 + iota
                tot = hist[pl.ds(v * 16, 16)]
                occ = (tot > 0) & (cellid < NCELL)
                return acc + occ.astype(jnp.int32)
            acc = jax.lax.fori_loop(0, CR // 16, occv,
                                    jnp.zeros((16,), jnp.int32))
            my_cnt = jnp.sum(acc, axis=0)
            small[...] = jnp.full((16,), my_cnt, jnp.int32)
            pltpu.sync_copy(small, sh_p.at[t, pl.ds(0, 16)])
            plsc.subcore_barrier()
            pltpu.sync_copy(sh_p, pbuf)
            cnts = plsc.load_gather(pbuf, [iota, jnp.zeros((16,), jnp.int32)])
            slot_start = jnp.sum(jnp.where(iota < t, cnts, 0), axis=0)
            total_occ = jnp.sum(cnts, axis=0)

            @pl.when(t == 0)
            def _():
                small[...] = jnp.full((16,), jnp.minimum(total_occ, MAXV),
                                      jnp.int32)
                pltpu.sync_copy(small, voxnum_hbm)
            return slot_start
        slot_start = pl.run_scoped(p3a, pltpu.VMEM((16, 128), jnp.int32),
                                   pltpu.VMEM((16,), jnp.int32))

        def p3b(slotq, u2, n2, z2, y2, x2, irow):
            slot_run = slot_start
            kept_tot = jnp.int32(0)
            for q in range(4):
                qbase = q * CRQ

                def vq(v, carry):
                    srun, kq = carry
                    cellid = lo + qbase + v * 16 + iota
                    tot = hist[pl.ds(qbase + v * 16, 16)]
                    occ = (tot > 0) & (cellid < NCELL)
                    occ_i = occ.astype(jnp.int32)
                    pref = plsc.cumsum(occ_i)
                    slotl = srun + pref - occ_i
                    kept = occ & (slotl < MAXV)
                    kept_i = kept.astype(jnp.int32)
                    slotq[pl.ds(v * 16, 16)] = jnp.where(
                        kept, slotl, jnp.int32(MAXV))
                    kpref = plsc.cumsum(kept_i)
                    pos = kq + kpref - kept_i
                    pidx = [pos // 128, pos % 128]
                    plsc.store_scatter(u2, pidx, cellid, mask=kept)
                    plsc.store_scatter(n2, pidx, jnp.minimum(tot, MAXP),
                                       mask=kept)
                    uz = cellid // (GX * GY)
                    rem = cellid % (GX * GY)
                    plsc.store_scatter(z2, pidx, uz, mask=kept)
                    plsc.store_scatter(y2, pidx, rem // GX, mask=kept)
                    plsc.store_scatter(x2, pidx, rem % GX, mask=kept)
                    return (srun + jnp.sum(occ_i, axis=0),
                            kq + jnp.sum(kept_i, axis=0))
                slot_run, kept_q = jax.lax.fori_loop(
                    0, CRQ // 16, vq, (slot_run, jnp.int32(0)))
                pltpu.sync_copy(slotq,
                                slot_hbm.at[pl.ds(t * CR + qbase, CRQ)])

                def srow(r, _):
                    for k in range(8):
                        prow = r * 128 + k * 16 + iota
                        tgt = slot_start + kept_tot + prow
                        dump = MAXV + ((prow + t * 283) & (TRASH - 1))
                        slot_or_dump = jnp.where(prow < kept_q, tgt, dump)
                        irow[0, pl.ds(k * 16, 16)] = slot_or_dump
                        irow[1, pl.ds(k * 16, 16)] = slot_or_dump * 3
                        irow[2, pl.ds(k * 16, 16)] = slot_or_dump * 3 + 1
                        irow[3, pl.ds(k * 16, 16)] = slot_or_dump * 3 + 2
                    pltpu.async_copy(u2.at[r], uniq_hbm.at[irow.at[0]],
                                     sem).wait()
                    pltpu.async_copy(n2.at[r], npts_hbm.at[irow.at[0]],
                                     sem).wait()
                    pltpu.async_copy(z2.at[r], coors_hbm.at[irow.at[1]],
                                     sem).wait()
                    pltpu.async_copy(y2.at[r], coors_hbm.at[irow.at[2]],
                                     sem).wait()
                    pltpu.async_copy(x2.at[r], coors_hbm.at[irow.at[3]],
                                     sem).wait()
                    return 0
                jax.lax.fori_loop(0, (kept_q + 127) // 128, srow, 0)
                kept_tot = kept_tot + kept_q
        pl.run_scoped(p3b, pltpu.VMEM((CRQ,), jnp.int32),
                      pltpu.VMEM((13, 128), jnp.int32),
                      pltpu.VMEM((13, 128), jnp.int32),
                      pltpu.VMEM((13, 128), jnp.int32),
                      pltpu.VMEM((13, 128), jnp.int32),
                      pltpu.VMEM((13, 128), jnp.int32),
                      pltpu.VMEM((4, 128), jnp.int32))


def _kernel_b(pts_hbm, cells_hbm, ranks_hbm, slot_hbm, vox_ref, dummy,
              slot_tab, pts, cellsb, ranksb, idx2, val2, sem):
    cid = jax.lax.axis_index("c")
    sid = jax.lax.axis_index("s")
    w = cid * 16 + sid
    iota = _IOTA()
    pltpu.sync_copy(slot_hbm, slot_tab)

    off = 0
    for n in B_SUBS:
        start = w * BCH + off
        pltpu.sync_copy(pts_hbm.at[pl.ds(start * 4, n * 4)],
                        pts.at[pl.ds(0, n * 4)])
        pltpu.sync_copy(cells_hbm.at[pl.ds(start, n)], cellsb.at[pl.ds(0, n)])
        pltpu.sync_copy(ranks_hbm.at[pl.ds(start, n)], ranksb.at[pl.ds(0, n)])
        nrows = (n + 127) // 128
        if n % 128:
            for k4 in range(4):
                for kk in range(8):
                    r = k4 * 4 + (nrows - 1)
                    dump = VROWS * 4 + (((kk * 16 + iota + w * 67)
                                         & (VTRASH - 1)) * 4 + k4)
                    idx2[r, pl.ds(kk * 16, 16)] = dump

        def vloop(v, _):
            ids = cellsb[pl.ds(v * 16, 16)]
            rk = ranksb[pl.ds(v * 16, 16)]
            slot = plsc.load_gather(slot_tab, [ids])
            keep = (slot < MAXV) & (rk < MAXP)
            spread = (v * 16 + iota + w * 101) & (VTRASH - 1)
            addr = jnp.where(keep, slot * MAXP + rk, VROWS + spread)
            pos = v * 16 + iota
            rows = pos // 128
            cols = pos % 128
            for k4 in range(4):
                pk = plsc.load_gather(pts, [pos * 4 + k4])
                plsc.store_scatter(val2, [k4 * 4 + rows, cols], pk)
                plsc.store_scatter(idx2, [k4 * 4 + rows, cols], addr * 4 + k4)
            return 0
        jax.lax.fori_loop(0, n // 16, vloop, 0)

        waits = []
        for k4 in range(4):
            for rr in range(nrows):
                r = k4 * 4 + rr
                waits.append(pltpu.async_copy(
                    val2.at[r], vox_ref.at[idx2.at[r]], sem))
        for d in waits:
            d.wait()
        off += n

    @pl.when(w == 0)
    def _():
        cellsb[pl.ds(0, 16)] = jnp.zeros((16,), jnp.int32)
        pltpu.sync_copy(cellsb.at[pl.ds(0, 16)], dummy)


_mesh = plsc.VectorSubcoreMesh(**_MESH)

_launch_a = functools.partial(
    pl.kernel,
    out_type=[
        jax.ShapeDtypeStruct((NPP,), jnp.int32),
        jax.ShapeDtypeStruct((NPP + RTRASH,), jnp.int32),
        jax.ShapeDtypeStruct((TH,), jnp.int32),
        jax.ShapeDtypeStruct((UNIQ_LEN,), jnp.int32),
        jax.ShapeDtypeStruct((UNIQ_LEN,), jnp.int32),
        jax.ShapeDtypeStruct((CLEN,), jnp.int32),
        jax.ShapeDtypeStruct((16,), jnp.int32),
    ],
    mesh=_mesh, compiler_params=_CP,
    scratch_types=[
        pltpu.VMEM((CR,), jnp.int32),
        pltpu.VMEM_SHARED((NPP,), jnp.int32),
        pltpu.VMEM_SHARED((16, 128), jnp.int32),
        pltpu.SemaphoreType.DMA,
    ])(_kernel_a)

_launch_b = functools.partial(
    pl.kernel,
    out_type=jax.ShapeDtypeStruct((16,), jnp.int32),
    mesh=_mesh, compiler_params=_CP,
    scratch_types=[
        pltpu.VMEM((TH,), jnp.int32),
        pltpu.VMEM((2048,), jnp.float32),
        pltpu.VMEM((512,), jnp.int32),
        pltpu.VMEM((512,), jnp.int32),
        pltpu.VMEM((16, 128), jnp.int32),
        pltpu.VMEM((16, 128), jnp.float32),
        pltpu.SemaphoreType.DMA,
    ])(_kernel_b)


def kernel(points):
    pts = jnp.concatenate(
        [points, jnp.zeros((NPP - NP_, points.shape[1]), points.dtype)],
        axis=0).reshape(-1)
    cells, ranks, slot_tab, uniq, npts, coors_f, voxnum = _launch_a(pts)
    vox_ref = jax.new_ref(jnp.zeros((VOXF,), jnp.float32))
    _launch_b(pts, cells, ranks, slot_tab, vox_ref)
    vox = vox_ref[...]
    voxels = vox[: VROWS * 4].reshape(MAXV, MAXP, 4)
    coors = coors_f[: 3 * MAXV].reshape(MAXV, 3)
    num_points = npts[:MAXV]
    voxel_num = voxnum[0]
    return voxels, coors, num_points, voxel_num
